# baseline (speedup 1.0000x reference)
import functools

import jax
import jax.numpy as jnp
from jax import lax
from jax.experimental import pallas as pl
from jax.experimental.pallas import tpu as pltpu
from jax.experimental.pallas import tpu_sc as plsc

DIM = 128
NUM_CORES = 2
NUM_SUBCORES = 16
NW = NUM_CORES * NUM_SUBCORES

CHUNK = 64
NBUF = 8
K = 4


@functools.partial(jax.jit, static_argnums=(2,))
def _gather_rows(ids_flat, table, n_rows):
    rows_per_w = n_rows // NW
    steps = rows_per_w // CHUNK
    assert steps % NBUF == 0 and steps // NBUF >= 2
    mesh = plsc.VectorSubcoreMesh(core_axis_name="c", subcore_axis_name="s")

    @functools.partial(
        pl.kernel,
        mesh=mesh,
        out_type=jax.ShapeDtypeStruct((n_rows, DIM), jnp.float32),
        scratch_types=[
            pltpu.VMEM((rows_per_w,), jnp.int32),
            pltpu.VMEM((NBUF, CHUNK, DIM), jnp.float32),
            pltpu.SemaphoreType.DMA((NBUF,)),
            pltpu.SemaphoreType.DMA((NBUF,)),
        ],
    )
    def body(ids_hbm, table_hbm, out_hbm, idx_v, rows_v, gsem, ssem):
        wid = lax.axis_index("s") * NUM_CORES + lax.axis_index("c")
        base = wid * rows_per_w
        pltpu.sync_copy(ids_hbm.at[pl.ds(base, rows_per_w)], idx_v)

        def gather(item, buf):
            return pltpu.make_async_copy(
                table_hbm.at[idx_v.at[pl.ds(item * CHUNK, CHUNK)]],
                rows_v.at[buf],
                gsem.at[buf],
            )

        def store(item, buf):
            return pltpu.make_async_copy(
                rows_v.at[buf],
                out_hbm.at[pl.ds(base + item * CHUNK, CHUNK)],
                ssem.at[buf],
            )

        def emit(g, b, do_gather, do_store_wait):
            if do_gather:
                bk = (b + K) % NBUF
                if do_store_wait:
                    store(g + K - NBUF, bk).wait()
                gather(g + K, bk).start()
            gather(g, b).wait()
            store(g, b).start()

        for i in range(K):
            gather(i, i).start()
        for g in range(NBUF):
            emit(g, g, True, g + K - NBUF >= 0)

        def outer(g0, carry):
            for j in range(NBUF):
                emit(g0 * NBUF + j, j, True, True)
            return carry

        lax.fori_loop(1, steps // NBUF - 1, outer, 0)

        for j in range(NBUF):
            g = steps - NBUF + j
            emit(g, j, g + K < steps, True)
        for j in range(NBUF):
            store(steps - NBUF + j, j).wait()

    return body(ids_flat, table)


def kernel(input_ids, embedding_weight):
    b, l = input_ids.shape
    n_rows = b * l
    out = _gather_rows(input_ids.reshape(n_rows), embedding_weight, n_rows)
    return out.reshape(b, l, DIM)


# speedup vs baseline: 1.0375x; 1.0375x over previous
import functools

import jax
import jax.numpy as jnp
from jax import lax
from jax.experimental import pallas as pl
from jax.experimental.pallas import tpu as pltpu
from jax.experimental.pallas import tpu_sc as plsc

DIM = 128
NUM_CORES = 2
NUM_SUBCORES = 16
NW = NUM_CORES * NUM_SUBCORES

CHUNK = 64
NBUF = 4
K = 2


@functools.partial(jax.jit, static_argnums=(2,))
def _gather_rows(ids_flat, table, n_rows):
    rows_per_w = n_rows // NW
    steps = rows_per_w // CHUNK
    pairs = steps // 2
    assert pairs % NBUF == 0 and pairs // NBUF >= 2
    mesh = plsc.VectorSubcoreMesh(core_axis_name="c", subcore_axis_name="s")

    @functools.partial(
        pl.kernel,
        mesh=mesh,
        out_type=jax.ShapeDtypeStruct((n_rows, DIM), jnp.float32),
        scratch_types=[
            pltpu.VMEM((rows_per_w,), jnp.int32),
            pltpu.VMEM((NBUF, CHUNK, DIM), jnp.float32),
            pltpu.VMEM((NBUF, CHUNK, DIM), jnp.float32),
            pltpu.VMEM_SHARED((NUM_SUBCORES * NBUF, CHUNK, DIM), jnp.float32),
            pltpu.SemaphoreType.DMA((NBUF,)),
            pltpu.SemaphoreType.DMA((NBUF,)),
            pltpu.SemaphoreType.DMA((NBUF,)),
            pltpu.SemaphoreType.DMA((NBUF,)),
            pltpu.SemaphoreType.DMA((NBUF,)),
        ],
    )
    def body(ids_hbm, table_hbm, out_hbm, idx_v, rows_v, rows2_v, spm,
             gsem, ssem, gsem2, csem2, ssem2):
        wid = lax.axis_index("s") * NUM_CORES + lax.axis_index("c")
        sid = lax.axis_index("s")
        base = wid * rows_per_w
        pltpu.sync_copy(ids_hbm.at[pl.ds(base, rows_per_w)], idx_v)

        def gather_e(p, buf):
            return pltpu.make_async_copy(
                table_hbm.at[idx_v.at[pl.ds(2 * p * CHUNK, CHUNK)]],
                rows_v.at[buf],
                gsem.at[buf],
            )

        def store_e(p, buf):
            return pltpu.make_async_copy(
                rows_v.at[buf],
                out_hbm.at[pl.ds(base + 2 * p * CHUNK, CHUNK)],
                ssem.at[buf],
            )

        def gather_o(p, buf):
            return pltpu.make_async_copy(
                table_hbm.at[idx_v.at[pl.ds((2 * p + 1) * CHUNK, CHUNK)]],
                rows2_v.at[buf],
                gsem2.at[buf],
            )

        def copy_o(buf):
            return pltpu.make_async_copy(
                rows2_v.at[buf],
                spm.at[sid * NBUF + buf],
                csem2.at[buf],
            )

        def store_o(p, buf):
            return pltpu.make_async_copy(
                spm.at[sid * NBUF + buf],
                out_hbm.at[pl.ds(base + (2 * p + 1) * CHUNK, CHUNK)],
                ssem2.at[buf],
            )

        def emit(p, b, do_gather, do_store_wait, do_prev):
            if do_gather:
                bk = (b + K) % NBUF
                if do_store_wait:
                    store_e(p + K - NBUF, bk).wait()
                    store_o(p + K - NBUF, bk).wait()
                gather_e(p + K, bk).start()
                gather_o(p + K, bk).start()
            gather_e(p, b).wait()
            store_e(p, b).start()
            gather_o(p, b).wait()
            copy_o(b).start()
            if do_prev:
                bp = (b - 1) % NBUF
                copy_o(bp).wait()
                store_o(p - 1, bp).start()

        for i in range(K):
            gather_e(i, i).start()
            gather_o(i, i).start()
        for p in range(NBUF):
            emit(p, p, True, p + K - NBUF >= 0, p >= 1)

        def outer(p0, carry):
            for j in range(NBUF):
                emit(p0 * NBUF + j, j, True, True, True)
            return carry

        lax.fori_loop(1, pairs // NBUF - 1, outer, 0)

        for j in range(NBUF):
            p = pairs - NBUF + j
            emit(p, j, p + K < pairs, True, True)
        copy_o(NBUF - 1).wait()
        store_o(pairs - 1, NBUF - 1).start()
        for j in range(NBUF):
            store_e(pairs - NBUF + j, j).wait()
            store_o(pairs - NBUF + j, j).wait()

    return body(ids_flat, table)


def kernel(input_ids, embedding_weight):
    b, l = input_ids.shape
    n_rows = b * l
    out = _gather_rows(input_ids.reshape(n_rows), embedding_weight, n_rows)
    return out.reshape(b, l, DIM)
